# Initial kernel scaffold; baseline (speedup 1.0000x reference)
import jax
import jax.numpy as jnp
from jax.experimental import pallas as pl


def kernel(img, grid):
    raise NotImplementedError("write your pallas kernel here")


# speedup vs baseline: 1.1035x; 1.1035x over previous
import functools

import jax
import jax.numpy as jnp
from jax import lax
from jax.experimental import pallas as pl
from jax.experimental.pallas import tpu as pltpu
from jax.experimental.pallas import tpu_sc as plsc

B, H, W, C = 4, 224, 224, 384
L = 16
NPIX = B * H * W
P = 32
CCH = C // L


def _sampler_body(img_hbm, xg_hbm, yg_hbm, out_hbm,
                  xv, yv, wa_v, wb_v, wc_v, wd_v,
                  ia_v, ib_v, ic_v, id_v,
                  rows_a, rows_b, rows_c, rows_d, out_v,
                  sem, osem):
    info = plsc.get_sparse_core_info()
    nw = info.num_cores * info.num_subcores
    wid = lax.axis_index("s") * info.num_cores + lax.axis_index("c")
    pix_per_w = NPIX // nw
    nchunks = pix_per_w // P
    tile_base = wid * pix_per_w
    row_base = (tile_base // (H * W)) * (H * W)

    xscale = jnp.float32(0.5 * (W - 2))
    yscale = jnp.float32(0.5 * (H - 2))

    def chunk_body(ci, _):
        gbase = tile_base + ci * P
        pltpu.sync_copy(xg_hbm.at[pl.ds(gbase, P)], xv)
        pltpu.sync_copy(yg_hbm.at[pl.ds(gbase, P)], yv)

        for k in range(P // L):
            sl = pl.ds(k * L, L)
            x = (xv[sl] + 1.0) * xscale
            y = (yv[sl] + 1.0) * yscale
            x0 = jnp.clip(x.astype(jnp.int32), 0, W - 1)
            y0 = jnp.clip(y.astype(jnp.int32), 0, H - 1)
            x1 = jnp.minimum(x0 + 1, W - 1)
            y1 = jnp.minimum(y0 + 1, H - 1)
            x0f = x0.astype(jnp.float32)
            x1f = x1.astype(jnp.float32)
            y0f = y0.astype(jnp.float32)
            y1f = y1.astype(jnp.float32)
            wa = (x1f - x) * (y1f - y)
            wb = (x1f - x) * (y - y0f)
            wc = (x - x0f) * (y1f - y)
            wd = (x - x0f) * (y - y0f)
            for s in range(L):
                p = k * L + s
                wa_v[p, :] = jnp.full((L,), wa[s])
                wb_v[p, :] = jnp.full((L,), wb[s])
                wc_v[p, :] = jnp.full((L,), wc[s])
                wd_v[p, :] = jnp.full((L,), wd[s])
            r0 = row_base + y0 * W
            r1 = row_base + y1 * W
            ia_v[sl] = r0 + x0
            ib_v[sl] = r1 + x0
            ic_v[sl] = r0 + x1
            id_v[sl] = r1 + x1

        cp_a = pltpu.async_copy(img_hbm.at[ia_v], rows_a, sem)
        cp_b = pltpu.async_copy(img_hbm.at[ib_v], rows_b, sem)
        cp_c = pltpu.async_copy(img_hbm.at[ic_v], rows_c, sem)
        cp_d = pltpu.async_copy(img_hbm.at[id_v], rows_d, sem)
        cp_a.wait()
        cp_b.wait()
        cp_c.wait()
        cp_d.wait()

        def pix_body(p, _):
            wa = wa_v[p, :]
            wb = wb_v[p, :]
            wc = wc_v[p, :]
            wd = wd_v[p, :]
            for c in range(CCH):
                cs = pl.ds(c * L, L)
                acc = wa * rows_a[p, cs]
                acc = acc + wb * rows_b[p, cs]
                acc = acc + wc * rows_c[p, cs]
                acc = acc + wd * rows_d[p, cs]
                out_v[p, cs] = acc
            return _

        lax.fori_loop(0, P, pix_body, 0)
        pltpu.async_copy(out_v, out_hbm.at[pl.ds(gbase, P)], osem).wait()
        return _

    lax.fori_loop(0, nchunks, chunk_body, 0)


@jax.jit
def kernel(img, grid):
    img_rows = img.reshape(NPIX, C)
    xg = grid[:, 0, :, :].reshape(NPIX)
    yg = grid[:, 1, :, :].reshape(NPIX)

    mesh = plsc.VectorSubcoreMesh(core_axis_name="c", subcore_axis_name="s")
    sampler = functools.partial(
        pl.kernel,
        mesh=mesh,
        out_type=jax.ShapeDtypeStruct((NPIX, C), jnp.float32),
        scratch_types=[
            pltpu.VMEM((P,), jnp.float32),
            pltpu.VMEM((P,), jnp.float32),
            pltpu.VMEM((P, L), jnp.float32),
            pltpu.VMEM((P, L), jnp.float32),
            pltpu.VMEM((P, L), jnp.float32),
            pltpu.VMEM((P, L), jnp.float32),
            pltpu.VMEM((P,), jnp.int32),
            pltpu.VMEM((P,), jnp.int32),
            pltpu.VMEM((P,), jnp.int32),
            pltpu.VMEM((P,), jnp.int32),
            pltpu.VMEM((P, C), jnp.float32),
            pltpu.VMEM((P, C), jnp.float32),
            pltpu.VMEM((P, C), jnp.float32),
            pltpu.VMEM((P, C), jnp.float32),
            pltpu.VMEM((P, C), jnp.float32),
            pltpu.SemaphoreType.DMA,
            pltpu.SemaphoreType.DMA,
        ],
    )(_sampler_body)
    out = sampler(img_rows, xg, yg)
    return out.reshape(B, H, W, C)


# speedup vs baseline: 1.6329x; 1.4798x over previous
import functools

import jax
import jax.numpy as jnp
from jax import lax
from jax.experimental import pallas as pl
from jax.experimental.pallas import tpu as pltpu
from jax.experimental.pallas import tpu_sc as plsc

B, H, W, C = 4, 224, 224, 384
L = 16
NPIX = B * H * W
P = 16
CCH = C // L


def _sampler_body(img_hbm, xg_hbm, yg_hbm, out_hbm,
                  xv, yv,
                  w_v, i_v, rows,
                  gsem0, gsem1, osem0, osem1):
    info = plsc.get_sparse_core_info()
    nw = info.num_cores * info.num_subcores
    wid = lax.axis_index("s") * info.num_cores + lax.axis_index("c")
    pix_per_w = NPIX // nw
    nchunks = pix_per_w // P
    tile_base = wid * pix_per_w
    row_base = (tile_base // (H * W)) * (H * W)

    xscale = jnp.float32(0.5 * (W - 2))
    yscale = jnp.float32(0.5 * (H - 2))
    gsems = (gsem0, gsem1)
    osems = (osem0, osem1)

    def compute_and_fire(ci, s, drain_out):
        pltpu.sync_copy(xg_hbm.at[pl.ds(tile_base + ci * P, P)], xv.at[s])
        pltpu.sync_copy(yg_hbm.at[pl.ds(tile_base + ci * P, P)], yv.at[s])
        for k in range(P // L):
            sl = pl.ds(k * L, L)
            x = (xv[s, sl] + 1.0) * xscale
            y = (yv[s, sl] + 1.0) * yscale
            x0 = jnp.clip(x.astype(jnp.int32), 0, W - 1)
            y0 = jnp.clip(y.astype(jnp.int32), 0, H - 1)
            x1 = jnp.minimum(x0 + 1, W - 1)
            y1 = jnp.minimum(y0 + 1, H - 1)
            x0f = x0.astype(jnp.float32)
            x1f = x1.astype(jnp.float32)
            y0f = y0.astype(jnp.float32)
            y1f = y1.astype(jnp.float32)
            wa = (x1f - x) * (y1f - y)
            wb = (x1f - x) * (y - y0f)
            wc = (x - x0f) * (y1f - y)
            wd = (x - x0f) * (y - y0f)
            for t in range(L):
                p = k * L + t
                w_v[s, 0, p, :] = jnp.full((L,), wa[t])
                w_v[s, 1, p, :] = jnp.full((L,), wb[t])
                w_v[s, 2, p, :] = jnp.full((L,), wc[t])
                w_v[s, 3, p, :] = jnp.full((L,), wd[t])
            r0 = row_base + y0 * W
            r1 = row_base + y1 * W
            i_v[s, 0, sl] = r0 + x0
            i_v[s, 1, sl] = r1 + x0
            i_v[s, 2, sl] = r0 + x1
            i_v[s, 3, sl] = r1 + x1
        for j in range(1, 4):
            pltpu.make_async_copy(
                img_hbm.at[i_v.at[s, j]], rows.at[s, j], gsems[s]).start()
        @pl.when(drain_out)
        def _drain():
            pltpu.make_async_copy(
                rows.at[s, 0], out_hbm.at[pl.ds(tile_base, P)],
                osems[s]).wait()
        pltpu.make_async_copy(
            img_hbm.at[i_v.at[s, 0]], rows.at[s, 0], gsems[s]).start()

    def combine_and_store(ci, s):
        for j in range(4):
            pltpu.make_async_copy(
                img_hbm.at[i_v.at[s, j]], rows.at[s, j], gsems[s]).wait()

        def pix_body(p, _):
            wa = w_v[s, 0, p, :]
            wb = w_v[s, 1, p, :]
            wc = w_v[s, 2, p, :]
            wd = w_v[s, 3, p, :]
            for c in range(CCH):
                cs = pl.ds(c * L, L)
                acc = wa * rows[s, 0, p, cs]
                acc = acc + wb * rows[s, 1, p, cs]
                acc = acc + wc * rows[s, 2, p, cs]
                acc = acc + wd * rows[s, 3, p, cs]
                rows[s, 0, p, cs] = acc
            return _

        lax.fori_loop(0, P, pix_body, 0)
        pltpu.make_async_copy(
            rows.at[s, 0], out_hbm.at[pl.ds(tile_base + ci * P, P)],
            osems[s]).start()

    compute_and_fire(0, 0, False)

    def pair_body(i, _):
        ci0 = 2 * i
        compute_and_fire(ci0 + 1, 1, i > 0)
        combine_and_store(ci0, 0)

        @pl.when(ci0 + 2 < nchunks)
        def _prefetch_next():
            compute_and_fire(ci0 + 2, 0, True)

        combine_and_store(ci0 + 1, 1)
        return _

    lax.fori_loop(0, nchunks // 2, pair_body, 0)
    for s in range(2):
        pltpu.make_async_copy(
            rows.at[s, 0], out_hbm.at[pl.ds(tile_base, P)], osems[s]).wait()


@jax.jit
def kernel(img, grid):
    img_rows = img.reshape(NPIX, C)
    xg = grid[:, 0, :, :].reshape(NPIX)
    yg = grid[:, 1, :, :].reshape(NPIX)

    mesh = plsc.VectorSubcoreMesh(core_axis_name="c", subcore_axis_name="s")
    sampler = functools.partial(
        pl.kernel,
        mesh=mesh,
        out_type=jax.ShapeDtypeStruct((NPIX, C), jnp.float32),
        scratch_types=[
            pltpu.VMEM((2, P), jnp.float32),
            pltpu.VMEM((2, P), jnp.float32),
            pltpu.VMEM((2, 4, P, L), jnp.float32),
            pltpu.VMEM((2, 4, P), jnp.int32),
            pltpu.VMEM((2, 4, P, C), jnp.float32),
            pltpu.SemaphoreType.DMA,
            pltpu.SemaphoreType.DMA,
            pltpu.SemaphoreType.DMA,
            pltpu.SemaphoreType.DMA,
        ],
    )(_sampler_body)
    out = sampler(img_rows, xg, yg)
    return out.reshape(B, H, W, C)
